# Initial kernel scaffold; baseline (speedup 1.0000x reference)
import jax
import jax.numpy as jnp
from jax.experimental import pallas as pl


def kernel(batch, x, edge_index, pseudo, W, root, bias, bn_gamma, bn_beta, bn_mean, bn_var, lin_W, lin_b):
    raise NotImplementedError("write your pallas kernel here")


# speedup vs baseline: 1.3187x; 1.3187x over previous
import functools

import jax
import jax.numpy as jnp
from jax import lax
from jax.experimental import pallas as pl
from jax.experimental.pallas import tpu as pltpu
from jax.experimental.pallas import tpu_sc as plsc

N_ = 10000
E_ = 160000
G_ = 64
DIM_ = 3
KK_ = 5
IN_C_ = 13
OUT_C_ = 64
S8_ = 8
KT_ = 125
CC_ = 80

NW_ = 32
EPW_ = E_ // NW_
CH_ = 40
NCH_ = EPW_ // CH_
NP_ = 10240
RPT_ = NP_ // 16
ZCH_ = 32


def _u_body(x_ref, w_ref, u_ref):
    u_ref[...] = jnp.dot(x_ref[...], w_ref[...],
                         preferred_element_type=jnp.float32)


def _compute_u(x, w_t):
    rb = 200
    return pl.pallas_call(
        _u_body,
        grid=(N_ // rb,),
        in_specs=[
            pl.BlockSpec((rb, IN_C_), lambda i: (i, 0)),
            pl.BlockSpec((IN_C_, KT_ * OUT_C_), lambda i: (0, 0)),
        ],
        out_specs=pl.BlockSpec((rb, KT_ * OUT_C_), lambda i: (i, 0)),
        out_shape=jax.ShapeDtypeStruct((N_, KT_ * OUT_C_), jnp.float32),
    )(x, w_t)


def _prep_body(ps_ref, src_ref, basis_ref, gidx_ref):
    ps = ps_ref[...]
    src = src_ref[...]
    v = ps * float(KK_ - 1)
    fli = v.astype(jnp.int32)
    frac = v - fli.astype(jnp.float32)
    for s in range(S8_):
        bprod = None
        widx = None
        for d in range(DIM_):
            bit = (s >> d) & 1
            fr = frac[:, d:d + 1]
            term = fr if bit else (1.0 - fr)
            bprod = term if bprod is None else bprod * term
            wi = (fli[:, d:d + 1] + bit) % KK_
            wterm = wi * (KK_ ** d)
            widx = wterm if widx is None else widx + wterm
        basis_ref[:, s:s + 1] = bprod
        gidx_ref[:, s:s + 1] = src * KT_ + widx


def _prep(pseudo, src2):
    eb = 2000
    return pl.pallas_call(
        _prep_body,
        grid=(E_ // eb,),
        in_specs=[
            pl.BlockSpec((eb, DIM_), lambda i: (i, 0)),
            pl.BlockSpec((eb, 1), lambda i: (i, 0)),
        ],
        out_specs=[
            pl.BlockSpec((eb, S8_), lambda i: (i, 0)),
            pl.BlockSpec((eb, S8_), lambda i: (i, 0)),
        ],
        out_shape=[
            jax.ShapeDtypeStruct((E_, S8_), jnp.float32),
            jax.ShapeDtypeStruct((E_, S8_), jnp.int32),
        ],
    )(pseudo, src2)


_MESH = plsc.VectorSubcoreMesh(core_axis_name="c", subcore_axis_name="s")


@functools.partial(
    pl.kernel,
    out_type=jax.ShapeDtypeStruct((2, NP_, CC_), jnp.float32),
    mesh=_MESH,
    compiler_params=pltpu.CompilerParams(use_tc_tiling_on_sc=False),
    scratch_types=[
        pltpu.VMEM_SHARED((NP_, CC_), jnp.float32),
        pltpu.VMEM((CH_ * S8_,), jnp.int32),
        pltpu.VMEM((CH_ * S8_ + 16,), jnp.float32),
        pltpu.VMEM((CH_,), jnp.int32),
        pltpu.VMEM((CH_ * S8_, OUT_C_), jnp.float32),
        pltpu.VMEM((CH_, CC_), jnp.float32),
        pltpu.VMEM((ZCH_, CC_), jnp.float32),
        pltpu.SemaphoreType.DMA,
    ],
)
def _sc_scatter(gidx_hbm, basis_hbm, dst_hbm, u_hbm, out_hbm,
                acc_sh, idx_v, bas_v, dst_v, rows_v, msg_v, zb_v, sem):
    cid = lax.axis_index("c")
    sid = lax.axis_index("s")
    wid = sid * 2 + cid

    zero16 = jnp.zeros((16,), jnp.float32)
    lane = lax.iota(jnp.int32, 16)
    colv = [lane + cg * 16 for cg in range(4)]
    cvec = jnp.where(lane == 0, 1.0, 0.0).astype(jnp.float32)

    for r in range(ZCH_):
        for cg in range(CC_ // 16):
            zb_v[r, pl.ds(cg * 16, 16)] = zero16
    rbase = sid * RPT_

    def zbody(z, carry):
        pltpu.sync_copy(zb_v, acc_sh.at[pl.ds(rbase + z * ZCH_, ZCH_)])
        return carry

    lax.fori_loop(0, RPT_ // ZCH_, zbody, 0)
    plsc.subcore_barrier()

    ebase0 = wid * EPW_

    def chunk(c, carry):
        eb = ebase0 + c * CH_
        gb = eb * S8_
        pltpu.sync_copy(gidx_hbm.at[pl.ds(gb, CH_ * S8_)], idx_v)
        pltpu.sync_copy(basis_hbm.at[pl.ds(gb, CH_ * S8_)],
                        bas_v.at[pl.ds(0, CH_ * S8_)])
        pltpu.sync_copy(dst_hbm.at[pl.ds(eb, CH_)], dst_v)
        copies = []
        for q in range(4):
            copies.append(pltpu.async_copy(
                u_hbm.at[idx_v.at[pl.ds(q * 80, 80)]],
                rows_v.at[pl.ds(q * 80, 80)], sem))
        for cp in copies:
            cp.wait()

        def edge(j, ecarry):
            j8 = j * S8_
            acc = [zero16, zero16, zero16, zero16]
            bvec = bas_v[pl.ds(j8, 16)]
            for s in range(S8_):
                b = jnp.full((16,), bvec[s], jnp.float32)
                for cg in range(4):
                    r = rows_v[j8 + s, pl.ds(cg * 16, 16)]
                    acc[cg] = acc[cg] + b * r
            for cg in range(4):
                msg_v[j, pl.ds(cg * 16, 16)] = acc[cg]
            msg_v[j, pl.ds(64, 16)] = cvec
            return ecarry

        lax.fori_loop(0, CH_, edge, 0)
        pltpu.sync_copy(msg_v, acc_sh.at[dst_v], add=True)
        return carry

    lax.fori_loop(0, NCH_, chunk, 0)
    plsc.subcore_barrier()

    pltpu.sync_copy(acc_sh.at[pl.ds(rbase, RPT_)],
                    out_hbm.at[cid, pl.ds(rbase, RPT_)])


def _epi_body(part_ref, x_ref, root_ref, bias_ref, batch_ref, gamma_ref,
              beta_ref, mean_ref, var_ref, lw_ref, lb_ref, out_ref):
    s = part_ref[0] + part_ref[1]
    aggsum = s[:N_, :OUT_C_]
    cnt = s[:N_, OUT_C_:OUT_C_ + 1]
    agg = aggsum / jnp.maximum(cnt, 1.0)
    out = agg + jnp.dot(x_ref[...], root_ref[...],
                        preferred_element_type=jnp.float32) + bias_ref[...]
    out = jnp.where(out > 0.0, out, jnp.exp(jnp.minimum(out, 0.0)) - 1.0)
    b_row = batch_ref[...]
    gid = lax.broadcasted_iota(jnp.int32, (G_, N_), 0)
    oh = (gid == b_row).astype(jnp.float32)
    pooled_sum = jnp.dot(oh, out, preferred_element_type=jnp.float32)
    gcnt = jnp.dot(oh, jnp.ones((N_, 1), jnp.float32),
                   preferred_element_type=jnp.float32)
    pooled = pooled_sum / jnp.maximum(gcnt, 1.0)
    h = ((pooled - mean_ref[...]) / jnp.sqrt(var_ref[...] + 1e-5)
         * gamma_ref[...] + beta_ref[...])
    n = jnp.dot(h, lw_ref[...], preferred_element_type=jnp.float32) \
        + lb_ref[...]
    norm = jnp.sqrt(jnp.sum(n * n, axis=1, keepdims=True))
    out_ref[...] = n / jnp.maximum(norm, 1e-12)


def _epilogue(part, x, root, bias, batch_r, gamma, beta, mean, var, lw, lb):
    return pl.pallas_call(
        _epi_body,
        out_shape=jax.ShapeDtypeStruct((G_, 3), jnp.float32),
    )(part, x, root, bias, batch_r, gamma, beta, mean, var, lw, lb)


def kernel(batch, x, edge_index, pseudo, W, root, bias, bn_gamma, bn_beta,
           bn_mean, bn_var, lin_W, lin_b):
    src = edge_index[0].astype(jnp.int32)
    dst = edge_index[1].astype(jnp.int32)
    w_t = W.transpose(1, 0, 2).reshape(IN_C_, KT_ * OUT_C_)

    u = _compute_u(x, w_t)
    basis, gidx = _prep(pseudo, src.reshape(E_, 1))

    part = _sc_scatter(gidx.reshape(E_ * S8_), basis.reshape(E_ * S8_),
                       dst, u.reshape(N_ * KT_, OUT_C_))

    return _epilogue(
        part, x, root, bias.reshape(1, OUT_C_),
        batch.astype(jnp.int32).reshape(1, N_),
        bn_gamma.reshape(1, OUT_C_), bn_beta.reshape(1, OUT_C_),
        bn_mean.reshape(1, OUT_C_), bn_var.reshape(1, OUT_C_),
        lin_W, lin_b.reshape(1, 3))


# speedup vs baseline: 1.4665x; 1.1121x over previous
import functools

import jax
import jax.numpy as jnp
from jax import lax
from jax.experimental import pallas as pl
from jax.experimental.pallas import tpu as pltpu
from jax.experimental.pallas import tpu_sc as plsc

N_ = 10000
E_ = 160000
G_ = 64
DIM_ = 3
KK_ = 5
IN_C_ = 13
OUT_C_ = 64
S8_ = 8
KT_ = 125
CC_ = 80

NW_ = 32
EPW_ = E_ // NW_
CH_ = 40
NCH_ = EPW_ // CH_
NP_ = 10240
RPT_ = NP_ // 16
ZCH_ = 32


def _u_body(x_ref, w_ref, u_ref):
    u_ref[...] = jnp.dot(x_ref[...], w_ref[...],
                         preferred_element_type=jnp.float32)


def _compute_u(x, w_t):
    rb = 200
    return pl.pallas_call(
        _u_body,
        grid=(N_ // rb,),
        in_specs=[
            pl.BlockSpec((rb, IN_C_), lambda i: (i, 0)),
            pl.BlockSpec((IN_C_, KT_ * OUT_C_), lambda i: (0, 0)),
        ],
        out_specs=pl.BlockSpec((rb, KT_ * OUT_C_), lambda i: (i, 0)),
        out_shape=jax.ShapeDtypeStruct((N_, KT_ * OUT_C_), jnp.float32),
    )(x, w_t)


def _prep_body(ps_ref, src_ref, basis_ref, gidx_ref):
    ps = ps_ref[...]
    src = src_ref[...]
    v = ps * float(KK_ - 1)
    fli = v.astype(jnp.int32)
    frac = v - fli.astype(jnp.float32)
    for s in range(S8_):
        bprod = None
        widx = None
        for d in range(DIM_):
            bit = (s >> d) & 1
            fr = frac[:, d:d + 1]
            term = fr if bit else (1.0 - fr)
            bprod = term if bprod is None else bprod * term
            wi = (fli[:, d:d + 1] + bit) % KK_
            wterm = wi * (KK_ ** d)
            widx = wterm if widx is None else widx + wterm
        basis_ref[:, s:s + 1] = bprod
        gidx_ref[:, s:s + 1] = src * KT_ + widx


def _prep(pseudo, src2):
    eb = 2000
    return pl.pallas_call(
        _prep_body,
        grid=(E_ // eb,),
        in_specs=[
            pl.BlockSpec((eb, DIM_), lambda i: (i, 0)),
            pl.BlockSpec((eb, 1), lambda i: (i, 0)),
        ],
        out_specs=[
            pl.BlockSpec((eb, S8_), lambda i: (i, 0)),
            pl.BlockSpec((eb, S8_), lambda i: (i, 0)),
        ],
        out_shape=[
            jax.ShapeDtypeStruct((E_, S8_), jnp.float32),
            jax.ShapeDtypeStruct((E_, S8_), jnp.int32),
        ],
    )(pseudo, src2)


_MESH = plsc.VectorSubcoreMesh(core_axis_name="c", subcore_axis_name="s")


_QG_ = 4
_QR_ = CH_ * S8_ // _QG_


@functools.partial(
    pl.kernel,
    out_type=jax.ShapeDtypeStruct((2, NP_, CC_), jnp.float32),
    mesh=_MESH,
    compiler_params=pltpu.CompilerParams(use_tc_tiling_on_sc=False),
    scratch_types=[
        pltpu.VMEM_SHARED((NP_, CC_), jnp.float32),
        pltpu.VMEM((CH_ * S8_,), jnp.int32),
        pltpu.VMEM((CH_ * S8_,), jnp.int32),
        pltpu.VMEM((CH_ * S8_ + 16,), jnp.float32),
        pltpu.VMEM((CH_,), jnp.int32),
        pltpu.VMEM((CH_ * S8_, OUT_C_), jnp.float32),
        pltpu.VMEM((CH_ * S8_, OUT_C_), jnp.float32),
        pltpu.VMEM((CH_, CC_), jnp.float32),
        pltpu.VMEM((ZCH_, CC_), jnp.float32),
        pltpu.SemaphoreType.DMA,
        pltpu.SemaphoreType.DMA,
    ],
)
def _sc_scatter(gidx_hbm, basis_hbm, dst_hbm, u_hbm, out_hbm,
                acc_sh, i0_v, i1_v, bas_v, dst_v, r0_v, r1_v, msg_v, zb_v,
                sem0, sem1):
    cid = lax.axis_index("c")
    sid = lax.axis_index("s")
    wid = sid * 2 + cid

    zero16 = jnp.zeros((16,), jnp.float32)
    lane = lax.iota(jnp.int32, 16)
    cvec = jnp.where(lane == 0, 1.0, 0.0).astype(jnp.float32)

    ebase0 = wid * EPW_

    def issue_gather(c, ibuf, rbuf, sem):
        pltpu.sync_copy(
            gidx_hbm.at[pl.ds((ebase0 + c * CH_) * S8_, CH_ * S8_)], ibuf)
        for q in range(_QG_):
            pltpu.async_copy(
                u_hbm.at[ibuf.at[pl.ds(q * _QR_, _QR_)]],
                rbuf.at[pl.ds(q * _QR_, _QR_)], sem)

    def wait_gather(ibuf, rbuf, sem):
        for q in range(_QG_):
            pltpu.make_async_copy(
                u_hbm.at[ibuf.at[pl.ds(q * _QR_, _QR_)]],
                rbuf.at[pl.ds(q * _QR_, _QR_)], sem).wait()

    issue_gather(0, i0_v, r0_v, sem0)

    for r in range(ZCH_):
        for cg in range(CC_ // 16):
            zb_v[r, pl.ds(cg * 16, 16)] = zero16
    rbase = sid * RPT_

    def zbody(z, carry):
        pltpu.sync_copy(zb_v, acc_sh.at[pl.ds(rbase + z * ZCH_, ZCH_)])
        return carry

    lax.fori_loop(0, RPT_ // ZCH_, zbody, 0)
    plsc.subcore_barrier()

    def half(c, ibuf, rbuf, sem, nibuf, nbuf, nsem, last):
        if not last:
            issue_gather(c + 1, nibuf, nbuf, nsem)
        eb = ebase0 + c * CH_
        pltpu.sync_copy(basis_hbm.at[pl.ds(eb * S8_, CH_ * S8_)],
                        bas_v.at[pl.ds(0, CH_ * S8_)])
        pltpu.sync_copy(dst_hbm.at[pl.ds(eb, CH_)], dst_v)
        wait_gather(ibuf, rbuf, sem)

        def edge(j, ecarry):
            j8 = j * S8_
            acc = [zero16, zero16, zero16, zero16]
            bvec = bas_v[pl.ds(j8, 16)]
            for s in range(S8_):
                b = jnp.full((16,), bvec[s], jnp.float32)
                for cg in range(4):
                    r = rbuf[j8 + s, pl.ds(cg * 16, 16)]
                    acc[cg] = acc[cg] + b * r
            for cg in range(4):
                msg_v[j, pl.ds(cg * 16, 16)] = acc[cg]
            msg_v[j, pl.ds(64, 16)] = cvec
            return ecarry

        lax.fori_loop(0, CH_, edge, 0)
        pltpu.sync_copy(msg_v, acc_sh.at[dst_v], add=True)

    def pair(p, carry):
        c0 = p * 2
        half(c0, i0_v, r0_v, sem0, i1_v, r1_v, sem1, False)
        half(c0 + 1, i1_v, r1_v, sem1, i0_v, r0_v, sem0, False)
        return carry

    lax.fori_loop(0, (NCH_ - 1) // 2, pair, 0)
    half(NCH_ - 1, i0_v, r0_v, sem0, None, None, None, True)

    plsc.subcore_barrier()

    pltpu.sync_copy(acc_sh.at[pl.ds(rbase, RPT_)],
                    out_hbm.at[cid, pl.ds(rbase, RPT_)])


def _epi_body(part_ref, x_ref, root_ref, bias_ref, batch_ref, gamma_ref,
              beta_ref, mean_ref, var_ref, lw_ref, lb_ref, out_ref):
    s = part_ref[0] + part_ref[1]
    aggsum = s[:N_, :OUT_C_]
    cnt = s[:N_, OUT_C_:OUT_C_ + 1]
    agg = aggsum / jnp.maximum(cnt, 1.0)
    out = agg + jnp.dot(x_ref[...], root_ref[...],
                        preferred_element_type=jnp.float32) + bias_ref[...]
    out = jnp.where(out > 0.0, out, jnp.exp(jnp.minimum(out, 0.0)) - 1.0)
    b_row = batch_ref[...]
    gid = lax.broadcasted_iota(jnp.int32, (G_, N_), 0)
    oh = (gid == b_row).astype(jnp.float32)
    pooled_sum = jnp.dot(oh, out, preferred_element_type=jnp.float32)
    gcnt = jnp.dot(oh, jnp.ones((N_, 1), jnp.float32),
                   preferred_element_type=jnp.float32)
    pooled = pooled_sum / jnp.maximum(gcnt, 1.0)
    h = ((pooled - mean_ref[...]) / jnp.sqrt(var_ref[...] + 1e-5)
         * gamma_ref[...] + beta_ref[...])
    n = jnp.dot(h, lw_ref[...], preferred_element_type=jnp.float32) \
        + lb_ref[...]
    norm = jnp.sqrt(jnp.sum(n * n, axis=1, keepdims=True))
    out_ref[...] = n / jnp.maximum(norm, 1e-12)


def _epilogue(part, x, root, bias, batch_r, gamma, beta, mean, var, lw, lb):
    return pl.pallas_call(
        _epi_body,
        out_shape=jax.ShapeDtypeStruct((G_, 3), jnp.float32),
    )(part, x, root, bias, batch_r, gamma, beta, mean, var, lw, lb)


def kernel(batch, x, edge_index, pseudo, W, root, bias, bn_gamma, bn_beta,
           bn_mean, bn_var, lin_W, lin_b):
    src = edge_index[0].astype(jnp.int32)
    dst = edge_index[1].astype(jnp.int32)
    w_t = W.transpose(1, 0, 2).reshape(IN_C_, KT_ * OUT_C_)

    u = _compute_u(x, w_t)
    basis, gidx = _prep(pseudo, src.reshape(E_, 1))

    part = _sc_scatter(gidx.reshape(E_ * S8_), basis.reshape(E_ * S8_),
                       dst, u.reshape(N_ * KT_, OUT_C_))

    return _epilogue(
        part, x, root, bias.reshape(1, OUT_C_),
        batch.astype(jnp.int32).reshape(1, N_),
        bn_gamma.reshape(1, OUT_C_), bn_beta.reshape(1, OUT_C_),
        bn_mean.reshape(1, OUT_C_), bn_var.reshape(1, OUT_C_),
        lin_W, lin_b.reshape(1, 3))


# speedup vs baseline: 1.4690x; 1.0017x over previous
import functools

import jax
import jax.numpy as jnp
from jax import lax
from jax.experimental import pallas as pl
from jax.experimental.pallas import tpu as pltpu
from jax.experimental.pallas import tpu_sc as plsc

N_ = 10000
E_ = 160000
G_ = 64
DIM_ = 3
KK_ = 5
IN_C_ = 13
OUT_C_ = 64
S8_ = 8
KT_ = 125
CC_ = 80

NW_ = 32
EPW_ = E_ // NW_
CH_ = 40
NCH_ = EPW_ // CH_
NP_ = 10240
RPT_ = NP_ // 16
ZCH_ = 32


def _u_body(x_ref, w_ref, u_ref):
    u_ref[...] = jnp.dot(x_ref[...], w_ref[...],
                         preferred_element_type=jnp.float32)


def _compute_u(x, w_t):
    rb = 200
    return pl.pallas_call(
        _u_body,
        grid=(N_ // rb,),
        in_specs=[
            pl.BlockSpec((rb, IN_C_), lambda i: (i, 0)),
            pl.BlockSpec((IN_C_, KT_ * OUT_C_), lambda i: (0, 0)),
        ],
        out_specs=pl.BlockSpec((rb, KT_ * OUT_C_), lambda i: (i, 0)),
        out_shape=jax.ShapeDtypeStruct((N_, KT_ * OUT_C_), jnp.float32),
    )(x, w_t)


def _prep_body(ps_ref, src_ref, basis_ref, gidx_ref):
    ps = ps_ref[...]
    src = src_ref[...]
    v = ps * float(KK_ - 1)
    fli = v.astype(jnp.int32)
    frac = v - fli.astype(jnp.float32)
    for s in range(S8_):
        bprod = None
        widx = None
        for d in range(DIM_):
            bit = (s >> d) & 1
            fr = frac[:, d:d + 1]
            term = fr if bit else (1.0 - fr)
            bprod = term if bprod is None else bprod * term
            wi = (fli[:, d:d + 1] + bit) % KK_
            wterm = wi * (KK_ ** d)
            widx = wterm if widx is None else widx + wterm
        basis_ref[:, s:s + 1] = bprod
        gidx_ref[:, s:s + 1] = src * KT_ + widx


def _prep(pseudo, src2):
    eb = 2000
    return pl.pallas_call(
        _prep_body,
        grid=(E_ // eb,),
        in_specs=[
            pl.BlockSpec((eb, DIM_), lambda i: (i, 0)),
            pl.BlockSpec((eb, 1), lambda i: (i, 0)),
        ],
        out_specs=[
            pl.BlockSpec((eb, S8_), lambda i: (i, 0)),
            pl.BlockSpec((eb, S8_), lambda i: (i, 0)),
        ],
        out_shape=[
            jax.ShapeDtypeStruct((E_, S8_), jnp.float32),
            jax.ShapeDtypeStruct((E_, S8_), jnp.int32),
        ],
    )(pseudo, src2)


_MESH = plsc.VectorSubcoreMesh(core_axis_name="c", subcore_axis_name="s")


_QG_ = 4
_QR_ = CH_ * S8_ // _QG_


@functools.partial(
    pl.kernel,
    out_type=jax.ShapeDtypeStruct((2, NP_, CC_), jnp.float32),
    mesh=_MESH,
    compiler_params=pltpu.CompilerParams(use_tc_tiling_on_sc=False),
    scratch_types=[
        pltpu.VMEM_SHARED((NP_, CC_), jnp.float32),
        pltpu.VMEM((CH_ * S8_,), jnp.int32),
        pltpu.VMEM((CH_ * S8_,), jnp.int32),
        pltpu.VMEM((CH_ * S8_ + 16,), jnp.float32),
        pltpu.VMEM((CH_,), jnp.int32),
        pltpu.VMEM((CH_ * S8_, OUT_C_), jnp.float32),
        pltpu.VMEM((CH_ * S8_, OUT_C_), jnp.float32),
        pltpu.VMEM((CH_, CC_), jnp.float32),
        pltpu.VMEM((ZCH_, CC_), jnp.float32),
        pltpu.SemaphoreType.DMA,
        pltpu.SemaphoreType.DMA,
    ],
)
def _sc_scatter(gidx_hbm, basis_hbm, dst_hbm, u_hbm, out_hbm,
                acc_sh, i0_v, i1_v, bas_v, dst_v, r0_v, r1_v, msg_v, zb_v,
                sem0, sem1):
    cid = lax.axis_index("c")
    sid = lax.axis_index("s")
    wid = sid * 2 + cid

    zero16 = jnp.zeros((16,), jnp.float32)
    lane = lax.iota(jnp.int32, 16)
    cvec = jnp.where(lane == 0, 1.0, 0.0).astype(jnp.float32)
    splat_s = [jnp.full((16, 1), s, jnp.int32) for s in range(S8_)]
    _gdn = lax.GatherDimensionNumbers(
        offset_dims=(), collapsed_slice_dims=(0,), start_index_map=(0,))

    def _bsplat(bvec, s):
        return lax.gather(bvec, splat_s[s], _gdn, (1,),
                          mode=lax.GatherScatterMode.PROMISE_IN_BOUNDS)

    ebase0 = wid * EPW_

    for j0 in range(CH_):
        msg_v[j0, pl.ds(64, 16)] = cvec

    def issue_gather(c, ibuf, rbuf, sem):
        pltpu.sync_copy(
            gidx_hbm.at[pl.ds((ebase0 + c * CH_) * S8_, CH_ * S8_)], ibuf)
        for q in range(_QG_):
            pltpu.async_copy(
                u_hbm.at[ibuf.at[pl.ds(q * _QR_, _QR_)]],
                rbuf.at[pl.ds(q * _QR_, _QR_)], sem)

    def wait_gather(ibuf, rbuf, sem):
        for q in range(_QG_):
            pltpu.make_async_copy(
                u_hbm.at[ibuf.at[pl.ds(q * _QR_, _QR_)]],
                rbuf.at[pl.ds(q * _QR_, _QR_)], sem).wait()

    issue_gather(0, i0_v, r0_v, sem0)

    for r in range(ZCH_):
        for cg in range(CC_ // 16):
            zb_v[r, pl.ds(cg * 16, 16)] = zero16
    rbase = sid * RPT_

    def zbody(z, carry):
        pltpu.sync_copy(zb_v, acc_sh.at[pl.ds(rbase + z * ZCH_, ZCH_)])
        return carry

    lax.fori_loop(0, RPT_ // ZCH_, zbody, 0)
    plsc.subcore_barrier()

    def half(c, ibuf, rbuf, sem, nibuf, nbuf, nsem, last):
        if not last:
            issue_gather(c + 1, nibuf, nbuf, nsem)
        eb = ebase0 + c * CH_
        pltpu.sync_copy(basis_hbm.at[pl.ds(eb * S8_, CH_ * S8_)],
                        bas_v.at[pl.ds(0, CH_ * S8_)])
        pltpu.sync_copy(dst_hbm.at[pl.ds(eb, CH_)], dst_v)
        wait_gather(ibuf, rbuf, sem)

        def edge(jq, ecarry):
            for u in range(4):
                j = jq * 4 + u
                j8 = j * S8_
                acc = [zero16, zero16, zero16, zero16]
                bvec = bas_v[pl.ds(j8, 16)]
                for s in range(S8_):
                    b = _bsplat(bvec, s)
                    for cg in range(4):
                        r = rbuf[j8 + s, pl.ds(cg * 16, 16)]
                        acc[cg] = acc[cg] + b * r
                for cg in range(4):
                    msg_v[j, pl.ds(cg * 16, 16)] = acc[cg]
            return ecarry

        lax.fori_loop(0, CH_ // 4, edge, 0)
        pltpu.sync_copy(msg_v, acc_sh.at[dst_v], add=True)

    def pair(p, carry):
        c0 = p * 2
        half(c0, i0_v, r0_v, sem0, i1_v, r1_v, sem1, False)
        half(c0 + 1, i1_v, r1_v, sem1, i0_v, r0_v, sem0, False)
        return carry

    lax.fori_loop(0, (NCH_ - 1) // 2, pair, 0)
    half(NCH_ - 1, i0_v, r0_v, sem0, None, None, None, True)

    plsc.subcore_barrier()

    pltpu.sync_copy(acc_sh.at[pl.ds(rbase, RPT_)],
                    out_hbm.at[cid, pl.ds(rbase, RPT_)])


def _epi_body(part_ref, x_ref, root_ref, bias_ref, batch_ref, gamma_ref,
              beta_ref, mean_ref, var_ref, lw_ref, lb_ref, out_ref):
    s = part_ref[0] + part_ref[1]
    aggsum = s[:N_, :OUT_C_]
    cnt = s[:N_, OUT_C_:OUT_C_ + 1]
    agg = aggsum / jnp.maximum(cnt, 1.0)
    out = agg + jnp.dot(x_ref[...], root_ref[...],
                        preferred_element_type=jnp.float32) + bias_ref[...]
    out = jnp.where(out > 0.0, out, jnp.exp(jnp.minimum(out, 0.0)) - 1.0)
    b_row = batch_ref[...]
    gid = lax.broadcasted_iota(jnp.int32, (G_, N_), 0)
    oh = (gid == b_row).astype(jnp.float32)
    pooled_sum = jnp.dot(oh, out, preferred_element_type=jnp.float32)
    gcnt = jnp.dot(oh, jnp.ones((N_, 1), jnp.float32),
                   preferred_element_type=jnp.float32)
    pooled = pooled_sum / jnp.maximum(gcnt, 1.0)
    h = ((pooled - mean_ref[...]) / jnp.sqrt(var_ref[...] + 1e-5)
         * gamma_ref[...] + beta_ref[...])
    n = jnp.dot(h, lw_ref[...], preferred_element_type=jnp.float32) \
        + lb_ref[...]
    norm = jnp.sqrt(jnp.sum(n * n, axis=1, keepdims=True))
    out_ref[...] = n / jnp.maximum(norm, 1e-12)


def _epilogue(part, x, root, bias, batch_r, gamma, beta, mean, var, lw, lb):
    return pl.pallas_call(
        _epi_body,
        out_shape=jax.ShapeDtypeStruct((G_, 3), jnp.float32),
    )(part, x, root, bias, batch_r, gamma, beta, mean, var, lw, lb)


def kernel(batch, x, edge_index, pseudo, W, root, bias, bn_gamma, bn_beta,
           bn_mean, bn_var, lin_W, lin_b):
    src = edge_index[0].astype(jnp.int32)
    dst = edge_index[1].astype(jnp.int32)
    w_t = W.transpose(1, 0, 2).reshape(IN_C_, KT_ * OUT_C_)

    u = _compute_u(x, w_t)
    basis, gidx = _prep(pseudo, src.reshape(E_, 1))

    part = _sc_scatter(gidx.reshape(E_ * S8_), basis.reshape(E_ * S8_),
                       dst, u.reshape(N_ * KT_, OUT_C_))

    return _epilogue(
        part, x, root, bias.reshape(1, OUT_C_),
        batch.astype(jnp.int32).reshape(1, N_),
        bn_gamma.reshape(1, OUT_C_), bn_beta.reshape(1, OUT_C_),
        bn_mean.reshape(1, OUT_C_), bn_var.reshape(1, OUT_C_),
        lin_W, lin_b.reshape(1, 3))


# speedup vs baseline: 1.5967x; 1.0869x over previous
import functools

import jax
import jax.numpy as jnp
from jax import lax
from jax.experimental import pallas as pl
from jax.experimental.pallas import tpu as pltpu
from jax.experimental.pallas import tpu_sc as plsc

N_ = 10000
E_ = 160000
G_ = 64
DIM_ = 3
KK_ = 5
IN_C_ = 13
OUT_C_ = 64
S8_ = 8
KT_ = 125
CC_ = 80

NW_ = 32
EPW_ = E_ // NW_
CH_ = 40
NCH_ = EPW_ // CH_
NP_ = 10240
RPT_ = NP_ // 16
ZCH_ = 32


def _u_body(x_ref, w_ref, u_ref):
    u_ref[...] = jnp.dot(x_ref[...], w_ref[...],
                         preferred_element_type=jnp.float32)


def _compute_u(x, w_t):
    rb = 200
    return pl.pallas_call(
        _u_body,
        grid=(N_ // rb,),
        in_specs=[
            pl.BlockSpec((rb, IN_C_), lambda i: (i, 0)),
            pl.BlockSpec((IN_C_, KT_ * OUT_C_), lambda i: (0, 0)),
        ],
        out_specs=pl.BlockSpec((rb, KT_ * OUT_C_), lambda i: (i, 0)),
        out_shape=jax.ShapeDtypeStruct((N_, KT_ * OUT_C_), jnp.float32),
    )(x, w_t)


def _prep_body(ps_ref, src_ref, basis_ref, gidx_ref):
    ps = ps_ref[...]
    src = src_ref[...]
    v = ps * float(KK_ - 1)
    fli = v.astype(jnp.int32)
    frac = v - fli.astype(jnp.float32)
    for s in range(S8_):
        bprod = None
        widx = None
        for d in range(DIM_):
            bit = (s >> d) & 1
            fr = frac[:, d:d + 1]
            term = fr if bit else (1.0 - fr)
            bprod = term if bprod is None else bprod * term
            wi = (fli[:, d:d + 1] + bit) % KK_
            wterm = wi * (KK_ ** d)
            widx = wterm if widx is None else widx + wterm
        basis_ref[:, s:s + 1] = bprod
        gidx_ref[:, s:s + 1] = src * KT_ + widx


def _prep(pseudo, src2):
    eb = 2000
    return pl.pallas_call(
        _prep_body,
        grid=(E_ // eb,),
        in_specs=[
            pl.BlockSpec((eb, DIM_), lambda i: (i, 0)),
            pl.BlockSpec((eb, 1), lambda i: (i, 0)),
        ],
        out_specs=[
            pl.BlockSpec((eb, S8_), lambda i: (i, 0)),
            pl.BlockSpec((eb, S8_), lambda i: (i, 0)),
        ],
        out_shape=[
            jax.ShapeDtypeStruct((E_, S8_), jnp.float32),
            jax.ShapeDtypeStruct((E_, S8_), jnp.int32),
        ],
    )(pseudo, src2)


_MESH = plsc.VectorSubcoreMesh(core_axis_name="c", subcore_axis_name="s")


_QG_ = 4
_QR_ = CH_ * S8_ // _QG_


@functools.partial(
    pl.kernel,
    out_type=jax.ShapeDtypeStruct((2, NP_, CC_), jnp.float32),
    mesh=_MESH,
    compiler_params=pltpu.CompilerParams(use_tc_tiling_on_sc=False),
    scratch_types=[
        pltpu.VMEM_SHARED((NP_, CC_), jnp.float32),
        pltpu.VMEM((CH_ * S8_,), jnp.int32),
        pltpu.VMEM((CH_ * S8_,), jnp.int32),
        pltpu.VMEM((CH_ * S8_ + 16,), jnp.float32),
        pltpu.VMEM((CH_ * S8_ + 16,), jnp.float32),
        pltpu.VMEM((CH_,), jnp.int32),
        pltpu.VMEM((CH_,), jnp.int32),
        pltpu.VMEM((CH_ * S8_, OUT_C_), jnp.float32),
        pltpu.VMEM((CH_ * S8_, OUT_C_), jnp.float32),
        pltpu.VMEM((CH_, CC_), jnp.float32),
        pltpu.VMEM((ZCH_, CC_), jnp.float32),
        pltpu.SemaphoreType.DMA,
        pltpu.SemaphoreType.DMA,
    ],
)
def _sc_scatter(gidx_hbm, basis_hbm, dst_hbm, u_hbm, out_hbm,
                acc_sh, i0_v, i1_v, b0_v, b1_v, d0_v, d1_v, r0_v, r1_v,
                msg_v, zb_v, sem0, sem1):
    cid = lax.axis_index("c")
    sid = lax.axis_index("s")
    wid = sid * 2 + cid

    zero16 = jnp.zeros((16,), jnp.float32)
    lane = lax.iota(jnp.int32, 16)
    cvec = jnp.where(lane == 0, 1.0, 0.0).astype(jnp.float32)
    splat_s = [jnp.full((16, 1), s, jnp.int32) for s in range(S8_)]
    _gdn = lax.GatherDimensionNumbers(
        offset_dims=(), collapsed_slice_dims=(0,), start_index_map=(0,))

    def _bsplat(bvec, s):
        return lax.gather(bvec, splat_s[s], _gdn, (1,),
                          mode=lax.GatherScatterMode.PROMISE_IN_BOUNDS)

    ebase0 = wid * EPW_

    for j0 in range(CH_):
        msg_v[j0, pl.ds(64, 16)] = cvec

    def issue_gather(c, ibuf, bbuf, dbuf, rbuf, sem):
        eb = ebase0 + c * CH_
        pltpu.sync_copy(gidx_hbm.at[pl.ds(eb * S8_, CH_ * S8_)], ibuf)
        for q in range(_QG_):
            pltpu.async_copy(
                u_hbm.at[ibuf.at[pl.ds(q * _QR_, _QR_)]],
                rbuf.at[pl.ds(q * _QR_, _QR_)], sem)
        pltpu.async_copy(basis_hbm.at[pl.ds(eb * S8_, CH_ * S8_)],
                         bbuf.at[pl.ds(0, CH_ * S8_)], sem)
        pltpu.async_copy(dst_hbm.at[pl.ds(eb, CH_)], dbuf, sem)

    def wait_gather(ibuf, bbuf, dbuf, rbuf, sem):
        for q in range(_QG_):
            pltpu.make_async_copy(
                u_hbm.at[ibuf.at[pl.ds(q * _QR_, _QR_)]],
                rbuf.at[pl.ds(q * _QR_, _QR_)], sem).wait()
        pltpu.make_async_copy(basis_hbm.at[pl.ds(0, CH_ * S8_)],
                              bbuf.at[pl.ds(0, CH_ * S8_)], sem).wait()
        pltpu.make_async_copy(dst_hbm.at[pl.ds(0, CH_)], dbuf, sem).wait()

    issue_gather(0, i0_v, b0_v, d0_v, r0_v, sem0)

    for r in range(ZCH_):
        for cg in range(CC_ // 16):
            zb_v[r, pl.ds(cg * 16, 16)] = zero16
    rbase = sid * RPT_

    def zbody(z, carry):
        pltpu.sync_copy(zb_v, acc_sh.at[pl.ds(rbase + z * ZCH_, ZCH_)])
        return carry

    lax.fori_loop(0, RPT_ // ZCH_, zbody, 0)
    plsc.subcore_barrier()

    def half(c, bufs, nbufs, last):
        ibuf, bbuf, dbuf, rbuf, sem = bufs
        if not last:
            issue_gather(c + 1, *nbufs)
        wait_gather(ibuf, bbuf, dbuf, rbuf, sem)
        bas_v = bbuf
        dst_v = dbuf

        def edge(jq, ecarry):
            for u in range(4):
                j = jq * 4 + u
                j8 = j * S8_
                acc = [zero16, zero16, zero16, zero16]
                bvec = bas_v[pl.ds(j8, 16)]
                for s in range(S8_):
                    b = _bsplat(bvec, s)
                    for cg in range(4):
                        r = rbuf[j8 + s, pl.ds(cg * 16, 16)]
                        acc[cg] = acc[cg] + b * r
                for cg in range(4):
                    msg_v[j, pl.ds(cg * 16, 16)] = acc[cg]
            return ecarry

        lax.fori_loop(0, CH_ // 4, edge, 0)
        pltpu.sync_copy(msg_v, acc_sh.at[dst_v], add=True)

    bufs0 = (i0_v, b0_v, d0_v, r0_v, sem0)
    bufs1 = (i1_v, b1_v, d1_v, r1_v, sem1)

    def pair(p, carry):
        c0 = p * 2
        half(c0, bufs0, bufs1, False)
        half(c0 + 1, bufs1, bufs0, False)
        return carry

    lax.fori_loop(0, (NCH_ - 1) // 2, pair, 0)
    half(NCH_ - 1, bufs0, None, True)

    plsc.subcore_barrier()

    pltpu.sync_copy(acc_sh.at[pl.ds(rbase, RPT_)],
                    out_hbm.at[cid, pl.ds(rbase, RPT_)])


def _epi_body(part_ref, x_ref, root_ref, bias_ref, batch_ref, gamma_ref,
              beta_ref, mean_ref, var_ref, lw_ref, lb_ref, out_ref):
    s = part_ref[0] + part_ref[1]
    aggsum = s[:N_, :OUT_C_]
    cnt = s[:N_, OUT_C_:OUT_C_ + 1]
    agg = aggsum / jnp.maximum(cnt, 1.0)
    out = agg + jnp.dot(x_ref[...], root_ref[...],
                        preferred_element_type=jnp.float32) + bias_ref[...]
    out = jnp.where(out > 0.0, out, jnp.exp(jnp.minimum(out, 0.0)) - 1.0)
    b_row = batch_ref[...]
    gid = lax.broadcasted_iota(jnp.int32, (G_, N_), 0)
    oh = (gid == b_row).astype(jnp.float32)
    pooled_sum = jnp.dot(oh, out, preferred_element_type=jnp.float32)
    gcnt = jnp.dot(oh, jnp.ones((N_, 1), jnp.float32),
                   preferred_element_type=jnp.float32)
    pooled = pooled_sum / jnp.maximum(gcnt, 1.0)
    h = ((pooled - mean_ref[...]) / jnp.sqrt(var_ref[...] + 1e-5)
         * gamma_ref[...] + beta_ref[...])
    n = jnp.dot(h, lw_ref[...], preferred_element_type=jnp.float32) \
        + lb_ref[...]
    norm = jnp.sqrt(jnp.sum(n * n, axis=1, keepdims=True))
    out_ref[...] = n / jnp.maximum(norm, 1e-12)


def _epilogue(part, x, root, bias, batch_r, gamma, beta, mean, var, lw, lb):
    return pl.pallas_call(
        _epi_body,
        out_shape=jax.ShapeDtypeStruct((G_, 3), jnp.float32),
    )(part, x, root, bias, batch_r, gamma, beta, mean, var, lw, lb)


def kernel(batch, x, edge_index, pseudo, W, root, bias, bn_gamma, bn_beta,
           bn_mean, bn_var, lin_W, lin_b):
    src = edge_index[0].astype(jnp.int32)
    dst = edge_index[1].astype(jnp.int32)
    w_t = W.transpose(1, 0, 2).reshape(IN_C_, KT_ * OUT_C_)

    u = _compute_u(x, w_t)
    basis, gidx = _prep(pseudo, src.reshape(E_, 1))

    part = _sc_scatter(gidx.reshape(E_ * S8_), basis.reshape(E_ * S8_),
                       dst, u.reshape(N_ * KT_, OUT_C_))

    return _epilogue(
        part, x, root, bias.reshape(1, OUT_C_),
        batch.astype(jnp.int32).reshape(1, N_),
        bn_gamma.reshape(1, OUT_C_), bn_beta.reshape(1, OUT_C_),
        bn_mean.reshape(1, OUT_C_), bn_var.reshape(1, OUT_C_),
        lin_W, lin_b.reshape(1, 3))
